# baseline (speedup 1.0000x reference)
import jax
import jax.numpy as jnp
from jax.experimental import pallas as pl
from jax.experimental.pallas import tpu as pltpu

_G = 52
_GG = _G * _G
_C = 25
_STRIDE = 8.0
_NB = 4


def _body(anchor_ref, x_ref, out_ref):
    a = pl.program_id(1)
    p = jax.lax.broadcasted_iota(
        jnp.int32, (1, _GG), 1).astype(jnp.float32)
    gy = jnp.floor((p + 0.5) * (1.0 / _G))
    gx = p - _G * gy
    grid01 = jnp.concatenate([gx, gy], axis=0)

    def anchor_branch(ai):
        def br():
            sc23 = jnp.concatenate(
                [jnp.full((1, 1), anchor_ref[ai, 0], jnp.float32),
                 jnp.full((1, 1), anchor_ref[ai, 1], jnp.float32)], axis=0)
            for n in range(_NB):
                xa = x_ref[n, ai * _C:(ai + 1) * _C].reshape(_C, _GG)
                xy = (jax.nn.sigmoid(xa[0:2]) + grid01) * _STRIDE
                wh = jnp.exp(xa[2:4]) * sc23
                rest = jax.nn.sigmoid(xa[4:_C])
                y = jnp.concatenate([xy, wh, rest], axis=0)
                out_ref[n] = y.T
        return br

    jax.lax.switch(a, [anchor_branch(0), anchor_branch(1), anchor_branch(2)])


def kernel(x, anchor_wh):
    B = x.shape[0]
    out = pl.pallas_call(
        _body,
        grid=(B // _NB, 3),
        in_specs=[
            pl.BlockSpec(memory_space=pltpu.SMEM),
            pl.BlockSpec((_NB, 3 * _C, _G, _G), lambda b, a: (b, 0, 0, 0)),
        ],
        out_specs=pl.BlockSpec((_NB, _GG, _C), lambda b, a: (b, a, 0)),
        out_shape=jax.ShapeDtypeStruct((B, 3 * _GG, _C), jnp.float32),
        compiler_params=pltpu.CompilerParams(
            dimension_semantics=("arbitrary", "arbitrary"),
        ),
    )(anchor_wh, x)
    return out


# speedup vs baseline: 1.0305x; 1.0305x over previous
import jax
import jax.numpy as jnp
from jax.experimental import pallas as pl
from jax.experimental.pallas import tpu as pltpu

_G = 52
_GG = _G * _G
_C = 25
_STRIDE = 8.0
_NB = 4


def _body(anchor_ref, x_ref, out_hbm, vbuf, sems):
    step = pl.program_id(0)
    b0 = step * _NB
    p = jax.lax.broadcasted_iota(
        jnp.int32, (1, _GG), 1).astype(jnp.float32)
    gy = jnp.floor((p + 0.5) * (1.0 / _G))
    gx = p - _G * gy
    grid01 = jnp.concatenate([gx, gy], axis=0)

    def slot_copy(slot, n, a):
        return pltpu.make_async_copy(
            vbuf.at[slot],
            out_hbm.at[b0 + n, pl.ds(a * _GG, _GG), :],
            sems.at[slot])

    for n in range(_NB):
        for a in range(3):
            k = n * 3 + a
            slot = k % 2
            xa = x_ref[n, a * _C:(a + 1) * _C].reshape(_C, _GG)
            xy = (jax.nn.sigmoid(xa[0:2]) + grid01) * _STRIDE
            sc23 = jnp.concatenate(
                [jnp.full((1, 1), anchor_ref[a, 0], jnp.float32),
                 jnp.full((1, 1), anchor_ref[a, 1], jnp.float32)], axis=0)
            wh = jnp.exp(xa[2:4]) * sc23
            rest = jax.nn.sigmoid(xa[4:_C])
            y = jnp.concatenate([xy, wh, rest], axis=0)
            yt = y.T
            if k >= 2:
                slot_copy(slot, n, a).wait()
            else:
                @pl.when(step > 0)
                def _():
                    slot_copy(slot, n, a).wait()
            vbuf[slot] = yt
            slot_copy(slot, n, a).start()

    @pl.when(step == pl.num_programs(0) - 1)
    def _():
        slot_copy(0, 0, 0).wait()
        slot_copy(1, 0, 0).wait()


def kernel(x, anchor_wh):
    B = x.shape[0]
    out = pl.pallas_call(
        _body,
        grid=(B // _NB,),
        in_specs=[
            pl.BlockSpec(memory_space=pltpu.SMEM),
            pl.BlockSpec((_NB, 3 * _C, _G, _G), lambda b: (b, 0, 0, 0)),
        ],
        out_specs=pl.BlockSpec(memory_space=pl.ANY),
        out_shape=jax.ShapeDtypeStruct((B, 3 * _GG, _C), jnp.float32),
        scratch_shapes=[
            pltpu.VMEM((2, _GG, _C), jnp.float32),
            pltpu.SemaphoreType.DMA((2,)),
        ],
        compiler_params=pltpu.CompilerParams(
            dimension_semantics=("arbitrary",),
        ),
    )(anchor_wh, x)
    return out


# speedup vs baseline: 1.0440x; 1.0131x over previous
import jax
import jax.numpy as jnp
from jax.experimental import pallas as pl
from jax.experimental.pallas import tpu as pltpu

_G = 52
_GG = _G * _G
_C = 25
_STRIDE = 8.0
_NB = 4


def _body(anchor_ref, x_ref, out_hbm, vbuf, sems):
    step = pl.program_id(0)
    b0 = step * _NB
    p = jax.lax.broadcasted_iota(
        jnp.int32, (1, _GG), 1).astype(jnp.float32)
    gy = jnp.floor((p + 0.5) * (1.0 / _G))
    gx = p - _G * gy
    grid01 = jnp.concatenate([gx, gy], axis=0)

    def slot_copy(slot, n, a):
        return pltpu.make_async_copy(
            vbuf.at[slot],
            out_hbm.at[b0 + n, pl.ds(a * _GG, _GG), :],
            sems.at[slot])

    for n in range(_NB):
        for a in range(3):
            k = n * 3 + a
            slot = k % 4
            xa = x_ref[n, a * _C:(a + 1) * _C].reshape(_C, _GG)
            xy = (jax.nn.sigmoid(xa[0:2]) + grid01) * _STRIDE
            sc23 = jnp.concatenate(
                [jnp.full((1, 1), anchor_ref[a, 0], jnp.float32),
                 jnp.full((1, 1), anchor_ref[a, 1], jnp.float32)], axis=0)
            wh = jnp.exp(xa[2:4]) * sc23
            rest = jax.nn.sigmoid(xa[4:_C])
            y = jnp.concatenate([xy, wh, rest], axis=0)
            yt = y.T
            if k >= 4:
                slot_copy(slot, n, a).wait()
            else:
                @pl.when(step > 0)
                def _():
                    slot_copy(slot, n, a).wait()
            vbuf[slot] = yt
            slot_copy(slot, n, a).start()

    @pl.when(step == pl.num_programs(0) - 1)
    def _():
        slot_copy(0, 0, 0).wait()
        slot_copy(1, 0, 0).wait()
        slot_copy(2, 0, 0).wait()
        slot_copy(3, 0, 0).wait()


def kernel(x, anchor_wh):
    B = x.shape[0]
    out = pl.pallas_call(
        _body,
        grid=(B // _NB,),
        in_specs=[
            pl.BlockSpec(memory_space=pltpu.SMEM),
            pl.BlockSpec((_NB, 3 * _C, _G, _G), lambda b: (b, 0, 0, 0)),
        ],
        out_specs=pl.BlockSpec(memory_space=pl.ANY),
        out_shape=jax.ShapeDtypeStruct((B, 3 * _GG, _C), jnp.float32),
        scratch_shapes=[
            pltpu.VMEM((4, _GG, _C), jnp.float32),
            pltpu.SemaphoreType.DMA((4,)),
        ],
        compiler_params=pltpu.CompilerParams(
            dimension_semantics=("arbitrary",),
        ),
    )(anchor_wh, x)
    return out


# speedup vs baseline: 1.1368x; 1.0889x over previous
import jax
import jax.numpy as jnp
from jax.experimental import pallas as pl
from jax.experimental.pallas import tpu as pltpu

_G = 52
_GG = _G * _G
_C = 25
_STRIDE = 8.0
_NB = 4


def _body(anchor_ref, x_ref, out_ref):
    p = jax.lax.broadcasted_iota(
        jnp.int32, (1, _GG), 1).astype(jnp.float32)
    gy = jnp.floor((p + 0.5) * (1.0 / _G))
    gx = p - _G * gy
    grid01 = jnp.concatenate([gx, gy], axis=0)

    for n in range(_NB):
        for a in range(3):
            xa = x_ref[n, a * _C:(a + 1) * _C].reshape(_C, _GG)
            xy = (jax.nn.sigmoid(xa[0:2]) + grid01) * _STRIDE
            sc23 = jnp.concatenate(
                [jnp.full((1, 1), anchor_ref[a, 0], jnp.float32),
                 jnp.full((1, 1), anchor_ref[a, 1], jnp.float32)], axis=0)
            wh = jnp.exp(xa[2:4]) * sc23
            rest = jax.nn.sigmoid(xa[4:_C])
            y = jnp.concatenate([xy, wh, rest], axis=0)
            out_ref[n, a * _GG:(a + 1) * _GG, :] = y.T


def kernel(x, anchor_wh):
    B = x.shape[0]
    out = pl.pallas_call(
        _body,
        grid=(B // _NB,),
        in_specs=[
            pl.BlockSpec(memory_space=pltpu.SMEM),
            pl.BlockSpec((_NB, 3 * _C, _G, _G), lambda b: (b, 0, 0, 0)),
        ],
        out_specs=pl.BlockSpec((_NB, 3 * _GG, _C), lambda b: (b, 0, 0)),
        out_shape=jax.ShapeDtypeStruct((B, 3 * _GG, _C), jnp.float32),
        compiler_params=pltpu.CompilerParams(
            dimension_semantics=("arbitrary",),
        ),
    )(anchor_wh, x)
    return out
